# baseline (speedup 1.0000x reference)
import functools

import numpy as np
import jax
import jax.numpy as jnp
from jax import lax
from jax.experimental import pallas as pl
from jax.experimental.pallas import tpu as pltpu
from jax.experimental.pallas import tpu_sc as plsc

_B = 4096
_D = 13
_M = 64
_NT = 26
_V = 100000
_NI = 32
_NW = 32
_BPW = _B // _NW
_BT = 256
_GRID = _B // _BT


def _build_src():
    src = np.full((27, _NI), 351, np.int32)
    p = 0
    for i in range(27):
        for j in range(i):
            src[i, j] = p
            p += 1
    return src


_SRC = _build_src()


def _sc_body(idx_hbm, emb_hbm, out_hbm, idx_v, idx_sh, idx_s, rows_v, sem):
    sid = lax.axis_index("s")
    wid = sid * 2 + lax.axis_index("c")
    b0 = wid * _BPW

    def stage_idx(t):
        pltpu.sync_copy(idx_hbm.at[t, pl.ds(b0, _BPW)], idx_v)
        pltpu.sync_copy(idx_v, idx_sh.at[sid])
        pltpu.sync_copy(idx_sh.at[sid], idx_s.at[t % 2])

    stage_idx(0)

    def tbl(t, carry):
        buf = t % 2

        def row(k, carry2):
            r = idx_s[buf, k]
            pltpu.async_copy(emb_hbm.at[t, r], rows_v.at[buf, k], sem)
            return carry2

        lax.fori_loop(0, _BPW, row, 0, unroll=4)

        @pl.when(t + 1 < _NT)
        def _():
            stage_idx(t + 1)

        pltpu.make_async_copy(
            emb_hbm.at[0, pl.ds(0, _BPW)], rows_v.at[buf], sem).wait()
        pltpu.sync_copy(rows_v.at[buf], out_hbm.at[pl.ds(b0, _BPW), t + 1])
        return carry

    lax.fori_loop(0, _NT, tbl, 0)


def _sc_gather(idx, emb3):
    mesh = plsc.VectorSubcoreMesh(core_axis_name="c", subcore_axis_name="s")
    kern = functools.partial(
        pl.kernel,
        mesh=mesh,
        out_type=jax.ShapeDtypeStruct((_B, _NI, _M), jnp.float32),
        scratch_types=[
            pltpu.VMEM((_BPW,), jnp.int32),
            pltpu.MemorySpace.VMEM_SHARED((16, _BPW), jnp.int32),
            pltpu.SMEM((2, _BPW), jnp.int32),
            pltpu.VMEM((2, _BPW, _M), jnp.float32),
            pltpu.SemaphoreType.DMA,
        ],
    )(_sc_body)
    return kern(idx, emb3)


_TRC = 2048
_TRN = -(-_V // _TRC)


def _tr_body(in_ref, eye_ref, out_ref):
    x = in_ref[0]
    y = lax.dot_general(x.astype(jnp.bfloat16), eye_ref[...],
                        (((0,), (0,)), ((), ())),
                        preferred_element_type=jnp.float32)
    out_ref[0] = y


def _transpose_table(embT, eye_bf):
    return pl.pallas_call(
        _tr_body,
        grid=(_NT, _TRN),
        in_specs=[
            pl.BlockSpec((1, _M, _TRC), lambda t, c: (t, 0, c)),
            pl.BlockSpec((_M, _M), lambda t, c: (0, 0)),
        ],
        out_specs=pl.BlockSpec((1, _TRC, _M), lambda t, c: (t, c, 0)),
        out_shape=jax.ShapeDtypeStruct((_NT, _V, _M), jnp.float32),
    )(embT, eye_bf)


def _tc_body(dx_ref, ly_ref, w0t_ref, b0_ref, w1t_ref, b1_ref, w2t_ref, b2_ref,
             wxt_ref, sf_ref, b3_ref, w4t_ref, b4_ref, w5_ref, b5_ref, out_ref):
    x = dx_ref[...]
    x = jnp.maximum(jnp.dot(x, w0t_ref[...], preferred_element_type=jnp.float32)
                    + b0_ref[...], 0.0)
    x = jnp.maximum(jnp.dot(x, w1t_ref[...], preferred_element_type=jnp.float32)
                    + b1_ref[...], 0.0)
    x3 = jnp.maximum(jnp.dot(x, w2t_ref[...], preferred_element_type=jnp.float32)
                     + b2_ref[...], 0.0)

    ly = ly_ref[...]
    ii = lax.broadcasted_iota(jnp.int32, (_BT, _NI, _M), 1)
    t3 = jnp.where(ii == 0, x3[:, None, :], jnp.where(ii < 27, ly, 0.0))
    t3b = t3.astype(jnp.bfloat16)
    z = lax.dot_general(t3b, t3b, (((2,), (2,)), ((0,), (0,))),
                        preferred_element_type=jnp.float32)

    y = jnp.dot(x3, wxt_ref[...], preferred_element_type=jnp.float32) + b3_ref[...]
    sf = sf_ref[...]
    for i in range(1, 27):
        y = y + jnp.dot(z[:, i, :].astype(jnp.bfloat16), sf[i],
                        preferred_element_type=jnp.float32)

    t1 = jnp.maximum(y, 0.0)
    t2 = jnp.maximum(jnp.dot(t1, w4t_ref[...], preferred_element_type=jnp.float32)
                     + b4_ref[...], 0.0)
    logit = jnp.sum(t2 * w5_ref[...], axis=1, keepdims=True) + b5_ref[...]
    out_ref[...] = 1.0 / (1.0 + jnp.exp(-logit))


def _w_spec(shape):
    nd = len(shape)
    return pl.BlockSpec(shape, lambda i, _n=nd: (0,) * _n)


_TC_GRID = (_GRID,)
_TC_OUT_SHAPE = jax.ShapeDtypeStruct((_B, 1), jnp.float32)
_TC_IN_SPECS = [
    pl.BlockSpec((_BT, _D), lambda i: (i, 0)),
    pl.BlockSpec((_BT, _NI, _M), lambda i: (i, 0, 0)),
    _w_spec((_D, 512)),
    _w_spec((1, 512)),
    _w_spec((512, 256)),
    _w_spec((1, 256)),
    _w_spec((256, _M)),
    _w_spec((1, _M)),
    _w_spec((_M, 512)),
    _w_spec((27, _NI, 512)),
    _w_spec((1, 512)),
    _w_spec((512, 256)),
    _w_spec((1, 256)),
    _w_spec((1, 256)),
    _w_spec((1, 1)),
]
_TC_OUT_SPEC = pl.BlockSpec((_BT, 1), lambda i: (i, 0))


def _tc_call(interpret_args, *ops):
    return pl.pallas_call(
        _tc_body,
        grid=_TC_GRID,
        in_specs=_TC_IN_SPECS,
        out_specs=_TC_OUT_SPEC,
        out_shape=_TC_OUT_SHAPE,
        **interpret_args,
    )(*ops)


def kernel(dense_x, lS_o, lS_i, emb,
           bot_W0, bot_b0, bot_W1, bot_b1, bot_W2, bot_b2,
           top_W0, top_b0, top_W1, top_b1, top_W2, top_b2):
    idx = lS_i.astype(jnp.int32)
    embT = jnp.swapaxes(emb, 1, 2)
    eye_bf = jnp.eye(_M, dtype=jnp.bfloat16)
    emb_rm = _transpose_table(embT, eye_bf)
    ly3 = _sc_gather(idx, emb_rm)

    w0t = bot_W0.T
    w1t = bot_W1.T
    w2t = bot_W2.T
    wxt = top_W0[:, :_M].T
    wzt = top_W0[:, _M:].T
    wzt_pad = jnp.concatenate([wzt, jnp.zeros((1, 512), jnp.float32)], axis=0)
    sfold = jnp.take(wzt_pad, _SRC.reshape(-1), axis=0)
    sfold = sfold.reshape(27, _NI, 512).astype(jnp.bfloat16)
    w4t = top_W1.T

    ops = (dense_x, ly3,
           w0t, bot_b0[None, :], w1t, bot_b1[None, :], w2t, bot_b2[None, :],
           wxt, sfold, top_b0[None, :], w4t, top_b1[None, :],
           top_W2, top_b2[None, :])
    return _tc_call({}, *ops)


# speedup vs baseline: 1.0299x; 1.0299x over previous
import functools

import numpy as np
import jax
import jax.numpy as jnp
from jax import lax
from jax.experimental import pallas as pl
from jax.experimental.pallas import tpu as pltpu
from jax.experimental.pallas import tpu_sc as plsc

_B = 4096
_D = 13
_M = 64
_NT = 26
_V = 100000
_NI = 32
_NW = 32
_BPW = _B // _NW
_BT = 256
_GRID = _B // _BT


def _build_src():
    src = np.full((27, _NI), 351, np.int32)
    p = 0
    for i in range(27):
        for j in range(i):
            src[i, j] = p
            p += 1
    return src


_SRC = _build_src()


_TRC = 2048
_TRN = -(-_V // _TRC)


def _tr_body(in_ref, eye_ref, out_ref):
    x = in_ref[0]
    y = lax.dot_general(x.astype(jnp.bfloat16), eye_ref[...],
                        (((0,), (0,)), ((), ())),
                        preferred_element_type=jnp.float32)
    out_ref[0] = jnp.concatenate([y[:_TRC // 2], y[_TRC // 2:]], axis=1)


def _transpose_table(embT, eye_bf):
    return pl.pallas_call(
        _tr_body,
        grid=(_NT, _TRN),
        in_specs=[
            pl.BlockSpec((1, _M, _TRC), lambda t, c: (t, 0, c)),
            pl.BlockSpec((_M, _M), lambda t, c: (0, 0)),
        ],
        out_specs=pl.BlockSpec((1, _TRC // 2, 2 * _M), lambda t, c: (t, c, 0)),
        out_shape=jax.ShapeDtypeStruct((_NT, _V // 2, 2 * _M), jnp.float32),
    )(embT, eye_bf)


def _sc_body(idx_hbm, emb_hbm, out_hbm, idx_v, idx_sh, idx_s, rows_v, sem):
    sid = lax.axis_index("s")
    wid = sid * 2 + lax.axis_index("c")
    b0 = wid * _BPW

    def stage_idx(t):
        pltpu.sync_copy(idx_hbm.at[t, pl.ds(b0, _BPW)], idx_v)
        pltpu.sync_copy(idx_v, idx_sh.at[sid])
        pltpu.sync_copy(idx_sh.at[sid], idx_s.at[t % 2])

    stage_idx(0)

    def tbl(t, carry):
        buf = t % 2

        def row(k, carry2):
            r = idx_s[buf, k]
            rp = ((r >> 11) << 10) + (r & 1023)
            pltpu.async_copy(emb_hbm.at[t, rp], rows_v.at[buf, k], sem)
            return carry2

        lax.fori_loop(0, _BPW, row, 0, unroll=4)

        @pl.when(t + 1 < _NT)
        def _():
            stage_idx(t + 1)

        pltpu.make_async_copy(
            emb_hbm.at[0, pl.ds(0, _BPW)], rows_v.at[buf], sem).wait()
        pltpu.sync_copy(rows_v.at[buf], out_hbm.at[pl.ds(b0, _BPW), t + 1])
        return carry

    lax.fori_loop(0, _NT, tbl, 0)


def _sc_gather(idx, emb_p):
    mesh = plsc.VectorSubcoreMesh(core_axis_name="c", subcore_axis_name="s")
    kern = functools.partial(
        pl.kernel,
        mesh=mesh,
        out_type=jax.ShapeDtypeStruct((_B, _NI, 2 * _M), jnp.float32),
        scratch_types=[
            pltpu.VMEM((_BPW,), jnp.int32),
            pltpu.MemorySpace.VMEM_SHARED((16, _BPW), jnp.int32),
            pltpu.SMEM((2, _BPW), jnp.int32),
            pltpu.VMEM((2, _BPW, 2 * _M), jnp.float32),
            pltpu.SemaphoreType.DMA,
        ],
    )(_sc_body)
    return kern(idx, emb_p)


def _tc_body(dx_ref, ly_ref, q_ref, w0t_ref, b0_ref, w1t_ref, b1_ref,
             w2t_ref, b2_ref, wxt_ref, sf_ref, b3_ref, w4t_ref, b4_ref,
             w5_ref, b5_ref, out_ref):
    x = dx_ref[...]
    x = jnp.maximum(jnp.dot(x, w0t_ref[...], preferred_element_type=jnp.float32)
                    + b0_ref[...], 0.0)
    x = jnp.maximum(jnp.dot(x, w1t_ref[...], preferred_element_type=jnp.float32)
                    + b1_ref[...], 0.0)
    x3 = jnp.maximum(jnp.dot(x, w2t_ref[...], preferred_element_type=jnp.float32)
                     + b2_ref[...], 0.0)

    lyp = ly_ref[...]
    q = q_ref[...]
    ly = jnp.where(q == 0, lyp[:, :, :_M], lyp[:, :, _M:])

    ii = lax.broadcasted_iota(jnp.int32, (_BT, _NI, _M), 1)
    t3 = jnp.where(ii == 0, x3[:, None, :], jnp.where(ii < 27, ly, 0.0))
    t3b = t3.astype(jnp.bfloat16)
    z = lax.dot_general(t3b, t3b, (((2,), (2,)), ((0,), (0,))),
                        preferred_element_type=jnp.float32)

    y = jnp.dot(x3, wxt_ref[...], preferred_element_type=jnp.float32) + b3_ref[...]
    for i in range(1, 27):
        y = y + jnp.dot(z[:, i, :].astype(jnp.bfloat16), sf_ref[i],
                        preferred_element_type=jnp.float32)

    t1 = jnp.maximum(y, 0.0)
    t2 = jnp.maximum(jnp.dot(t1, w4t_ref[...], preferred_element_type=jnp.float32)
                     + b4_ref[...], 0.0)
    logit = jnp.sum(t2 * w5_ref[...], axis=1, keepdims=True) + b5_ref[...]
    out_ref[...] = 1.0 / (1.0 + jnp.exp(-logit))


def _w_spec(shape):
    nd = len(shape)
    return pl.BlockSpec(shape, lambda i, _n=nd: (0,) * _n)


_TC_GRID = (_GRID,)
_TC_OUT_SHAPE = jax.ShapeDtypeStruct((_B, 1), jnp.float32)
_TC_IN_SPECS = [
    pl.BlockSpec((_BT, _D), lambda i: (i, 0)),
    pl.BlockSpec((_BT, _NI, 2 * _M), lambda i: (i, 0, 0)),
    pl.BlockSpec((_BT, _NI, 1), lambda i: (i, 0, 0)),
    _w_spec((_D, 512)),
    _w_spec((1, 512)),
    _w_spec((512, 256)),
    _w_spec((1, 256)),
    _w_spec((256, _M)),
    _w_spec((1, _M)),
    _w_spec((_M, 512)),
    _w_spec((27, _NI, 512)),
    _w_spec((1, 512)),
    _w_spec((512, 256)),
    _w_spec((1, 256)),
    _w_spec((1, 256)),
    _w_spec((1, 1)),
]
_TC_OUT_SPEC = pl.BlockSpec((_BT, 1), lambda i: (i, 0))


def _tc_call(interpret_args, *ops):
    return pl.pallas_call(
        _tc_body,
        grid=_TC_GRID,
        in_specs=_TC_IN_SPECS,
        out_specs=_TC_OUT_SPEC,
        out_shape=_TC_OUT_SHAPE,
        **interpret_args,
    )(*ops)


def kernel(dense_x, lS_o, lS_i, emb,
           bot_W0, bot_b0, bot_W1, bot_b1, bot_W2, bot_b2,
           top_W0, top_b0, top_W1, top_b1, top_W2, top_b2):
    idx = lS_i.astype(jnp.int32)
    embT = jnp.swapaxes(emb, 1, 2)
    eye_bf = jnp.eye(_M, dtype=jnp.bfloat16)
    emb_p = _transpose_table(embT, eye_bf)
    ly5 = _sc_gather(idx, emb_p)

    q = jnp.zeros((_B, _NI), jnp.int32).at[:, 1:27].set(((idx >> 10) & 1).T)
    q = q[:, :, None]

    w0t = bot_W0.T
    w1t = bot_W1.T
    w2t = bot_W2.T
    wxt = top_W0[:, :_M].T
    wzt = top_W0[:, _M:].T
    wzt_pad = jnp.concatenate([wzt, jnp.zeros((1, 512), jnp.float32)], axis=0)
    sfold = jnp.take(wzt_pad, _SRC.reshape(-1), axis=0)
    sfold = sfold.reshape(27, _NI, 512).astype(jnp.bfloat16)
    w4t = top_W1.T

    ops = (dense_x, ly5, q,
           w0t, bot_b0[None, :], w1t, bot_b1[None, :], w2t, bot_b2[None, :],
           wxt, sfold, top_b0[None, :], w4t, top_b1[None, :],
           top_W2, top_b2[None, :])
    return _tc_call({}, *ops)


# speedup vs baseline: 2.7966x; 2.7154x over previous
import functools

import numpy as np
import jax
import jax.numpy as jnp
from jax import lax
from jax.experimental import pallas as pl
from jax.experimental.pallas import tpu as pltpu
from jax.experimental.pallas import tpu_sc as plsc

_B = 4096
_D = 13
_M = 64
_NT = 26
_V = 100000
_NW = 32
_NU = _NT * _M
_UPW = _NU // _NW
_BT = 256
_GRID = _B // _BT


def _pair_col(i, j):
    return i * (i - 1) // 2 + j


def _sc_body(idx_hbm, emb_hbm, out_hbm, idx_v, row_v, out_v, sem):
    wid = lax.axis_index("s") * 2 + lax.axis_index("c")

    def unit(j, carry):
        u = wid * _UPW + j
        t = u // _M
        d = u % _M
        pltpu.sync_copy(idx_hbm.at[t], idx_v)
        pltpu.sync_copy(emb_hbm.at[t, d], row_v)

        def grp(g, carry2):
            iv = idx_v[pl.ds(g * 16, 16)]
            out_v[pl.ds(g * 16, 16)] = plsc.load_gather(row_v, [iv])
            return carry2

        lax.fori_loop(0, _B // 16, grp, 0, unroll=4)
        pltpu.sync_copy(out_v, out_hbm.at[t, d])
        return carry

    lax.fori_loop(0, _UPW, unit, 0)


def _sc_gather(idx, embT):
    mesh = plsc.VectorSubcoreMesh(core_axis_name="c", subcore_axis_name="s")
    kern = functools.partial(
        pl.kernel,
        mesh=mesh,
        out_type=jax.ShapeDtypeStruct((_NT, _M, _B), jnp.float32),
        scratch_types=[
            pltpu.VMEM((_B,), jnp.int32),
            pltpu.VMEM((_V,), jnp.float32),
            pltpu.VMEM((_B,), jnp.float32),
            pltpu.SemaphoreType.DMA,
        ],
        compiler_params=pltpu.CompilerParams(needs_layout_passes=False),
    )(_sc_body)
    return kern(idx, embT)


def _tc_body(dx_ref, ly_ref, w0t_ref, b0_ref, w1t_ref, b1_ref,
             w2t_ref, b2_ref, eye_ref, wx_ref, wf_ref, b3_ref, w4_ref,
             b4_ref, w5_ref, b5_ref, out_ref, zp_scr):
    x = dx_ref[...]
    x = jnp.maximum(jnp.dot(x, w0t_ref[...], preferred_element_type=jnp.float32)
                    + b0_ref[...], 0.0)
    x = jnp.maximum(jnp.dot(x, w1t_ref[...], preferred_element_type=jnp.float32)
                    + b1_ref[...], 0.0)
    x3 = jnp.maximum(jnp.dot(x, w2t_ref[...], preferred_element_type=jnp.float32)
                     + b2_ref[...], 0.0)
    x3t = lax.dot_general(x3, eye_ref[...], (((0,), (0,)), ((), ())),
                          preferred_element_type=jnp.float32)

    for t in range(_NT):
        a = ly_ref[t]
        zx = jnp.sum(a * x3t, axis=0, keepdims=True)
        zp_scr[pl.ds(_pair_col(1 + t, 0), 1), :] = zx
        for u in range(t):
            b = ly_ref[u]
            z = jnp.sum(a * b, axis=0, keepdims=True)
            zp_scr[pl.ds(_pair_col(1 + t, 1 + u), 1), :] = z
    zp_scr[pl.ds(351, 33), :] = jnp.zeros((33, _BT), jnp.float32)

    y = lax.dot_general(wx_ref[...], x3t, (((1,), (0,)), ((), ())),
                        preferred_element_type=jnp.float32)
    y = y + lax.dot_general(wf_ref[...], zp_scr[...].astype(jnp.bfloat16),
                            (((1,), (0,)), ((), ())),
                            preferred_element_type=jnp.float32)
    t1 = jnp.maximum(y + b3_ref[...], 0.0)
    t2 = jnp.maximum(
        lax.dot_general(w4_ref[...], t1, (((1,), (0,)), ((), ())),
                        preferred_element_type=jnp.float32) + b4_ref[...], 0.0)
    logit = jnp.sum(t2 * w5_ref[...], axis=0, keepdims=True) + b5_ref[...]
    out_ref[...] = 1.0 / (1.0 + jnp.exp(-logit))


def _w_spec(shape):
    nd = len(shape)
    return pl.BlockSpec(shape, lambda i, _n=nd: (0,) * _n)


_TC_GRID = (_GRID,)
_TC_OUT_SHAPE = jax.ShapeDtypeStruct((1, _B), jnp.float32)
_TC_IN_SPECS = [
    pl.BlockSpec((_BT, _D), lambda i: (i, 0)),
    pl.BlockSpec((_NT, _M, _BT), lambda i: (0, 0, i)),
    _w_spec((_D, 512)),
    _w_spec((1, 512)),
    _w_spec((512, 256)),
    _w_spec((1, 256)),
    _w_spec((256, _M)),
    _w_spec((1, _M)),
    _w_spec((_BT, _BT)),
    _w_spec((512, _M)),
    _w_spec((512, 384)),
    _w_spec((512, 1)),
    _w_spec((256, 512)),
    _w_spec((256, 1)),
    _w_spec((256, 1)),
    _w_spec((1, 1)),
]
_TC_OUT_SPEC = pl.BlockSpec((1, _BT), lambda i: (0, i))
_TC_SCRATCH = [pltpu.VMEM((384, _BT), jnp.float32)]


def _tc_call(interpret_args, *ops):
    return pl.pallas_call(
        _tc_body,
        grid=_TC_GRID,
        in_specs=_TC_IN_SPECS,
        out_specs=_TC_OUT_SPEC,
        out_shape=_TC_OUT_SHAPE,
        scratch_shapes=_TC_SCRATCH,
        **interpret_args,
    )(*ops)


def kernel(dense_x, lS_o, lS_i, emb,
           bot_W0, bot_b0, bot_W1, bot_b1, bot_W2, bot_b2,
           top_W0, top_b0, top_W1, top_b1, top_W2, top_b2):
    idx = lS_i.astype(jnp.int32)
    embT = jnp.swapaxes(emb, 1, 2)
    lyd = _sc_gather(idx, embT)

    w0t = bot_W0.T
    w1t = bot_W1.T
    w2t = bot_W2.T
    eye = jnp.eye(_BT, dtype=jnp.float32)
    wx = top_W0[:, :_M]
    wf = jnp.concatenate(
        [top_W0[:, _M:], jnp.zeros((512, 384 - 351), jnp.float32)],
        axis=1).astype(jnp.bfloat16)
    w4 = top_W1

    ops = (dense_x, lyd,
           w0t, bot_b0[None, :], w1t, bot_b1[None, :], w2t, bot_b2[None, :],
           eye, wx, wf, top_b0[:, None], w4, top_b1[:, None],
           top_W2.reshape(256, 1), top_b2[None, :])
    p = _tc_call({}, *ops)
    return p.reshape(_B, 1)


# speedup vs baseline: 3.0558x; 1.0927x over previous
import functools

import numpy as np
import jax
import jax.numpy as jnp
from jax import lax
from jax.experimental import pallas as pl
from jax.experimental.pallas import tpu as pltpu
from jax.experimental.pallas import tpu_sc as plsc

_B = 4096
_D = 13
_M = 64
_NT = 26
_V = 100000
_NW = 32
_NU = _NT * _M
_UPW = _NU // _NW
_BT = 256
_GRID = _B // _BT


def _pair_col(i, j):
    return i * (i - 1) // 2 + j


def _sc_body(idx_hbm, emb_hbm, out_hbm, idx_v, row_v, out_v, sem):
    wid = lax.axis_index("s") * 2 + lax.axis_index("c")

    def unit(j, prev_t):
        u = wid * _UPW + j
        t = u // _M
        d = u % _M

        @pl.when(t != prev_t)
        def _():
            pltpu.sync_copy(idx_hbm.at[t], idx_v)

        pltpu.sync_copy(emb_hbm.at[t, d], row_v)

        def grp(g, carry2):
            iv = idx_v[pl.ds(g * 16, 16)]
            out_v[pl.ds(g * 16, 16)] = plsc.load_gather(row_v, [iv])
            return carry2

        lax.fori_loop(0, _B // 16, grp, 0, unroll=4)
        pltpu.sync_copy(out_v, out_hbm.at[t, d])
        return t

    lax.fori_loop(0, _UPW, unit, -1)


def _sc_gather(idx, embT):
    mesh = plsc.VectorSubcoreMesh(core_axis_name="c", subcore_axis_name="s")
    kern = functools.partial(
        pl.kernel,
        mesh=mesh,
        out_type=jax.ShapeDtypeStruct((_NT, _M, _B), jnp.float32),
        scratch_types=[
            pltpu.VMEM((_B,), jnp.int32),
            pltpu.VMEM((_V,), jnp.float32),
            pltpu.VMEM((_B,), jnp.float32),
            pltpu.SemaphoreType.DMA,
        ],
        compiler_params=pltpu.CompilerParams(needs_layout_passes=False),
    )(_sc_body)
    return kern(idx, embT)


def _tc_body(dx_ref, ly_ref, w0t_ref, b0_ref, w1t_ref, b1_ref,
             w2t_ref, b2_ref, eye_ref, wx_ref, wf_ref, b3_ref, w4_ref,
             b4_ref, w5_ref, b5_ref, out_ref, zp_scr):
    x = dx_ref[...]
    x = jnp.maximum(jnp.dot(x, w0t_ref[...], preferred_element_type=jnp.float32)
                    + b0_ref[...], 0.0)
    x = jnp.maximum(jnp.dot(x, w1t_ref[...], preferred_element_type=jnp.float32)
                    + b1_ref[...], 0.0)
    x3 = jnp.maximum(jnp.dot(x, w2t_ref[...], preferred_element_type=jnp.float32)
                     + b2_ref[...], 0.0)
    x3t = lax.dot_general(x3, eye_ref[...], (((0,), (0,)), ((), ())),
                          preferred_element_type=jnp.float32)

    for t in range(_NT):
        a = ly_ref[t]
        zx = jnp.sum(a * x3t, axis=0, keepdims=True)
        zp_scr[pl.ds(_pair_col(1 + t, 0), 1), :] = zx
        for u in range(t):
            b = ly_ref[u]
            z = jnp.sum(a * b, axis=0, keepdims=True)
            zp_scr[pl.ds(_pair_col(1 + t, 1 + u), 1), :] = z
    zp_scr[pl.ds(351, 33), :] = jnp.zeros((33, _BT), jnp.float32)

    y = lax.dot_general(wx_ref[...], x3t, (((1,), (0,)), ((), ())),
                        preferred_element_type=jnp.float32)
    y = y + lax.dot_general(wf_ref[...], zp_scr[...].astype(jnp.bfloat16),
                            (((1,), (0,)), ((), ())),
                            preferred_element_type=jnp.float32)
    t1 = jnp.maximum(y + b3_ref[...], 0.0)
    t2 = jnp.maximum(
        lax.dot_general(w4_ref[...], t1, (((1,), (0,)), ((), ())),
                        preferred_element_type=jnp.float32) + b4_ref[...], 0.0)
    logit = jnp.sum(t2 * w5_ref[...], axis=0, keepdims=True) + b5_ref[...]
    out_ref[...] = 1.0 / (1.0 + jnp.exp(-logit))


def _w_spec(shape):
    nd = len(shape)
    return pl.BlockSpec(shape, lambda i, _n=nd: (0,) * _n)


_TC_GRID = (_GRID,)
_TC_OUT_SHAPE = jax.ShapeDtypeStruct((1, _B), jnp.float32)
_TC_IN_SPECS = [
    pl.BlockSpec((_BT, _D), lambda i: (i, 0)),
    pl.BlockSpec((_NT, _M, _BT), lambda i: (0, 0, i)),
    _w_spec((_D, 512)),
    _w_spec((1, 512)),
    _w_spec((512, 256)),
    _w_spec((1, 256)),
    _w_spec((256, _M)),
    _w_spec((1, _M)),
    _w_spec((_BT, _BT)),
    _w_spec((512, _M)),
    _w_spec((512, 384)),
    _w_spec((512, 1)),
    _w_spec((256, 512)),
    _w_spec((256, 1)),
    _w_spec((256, 1)),
    _w_spec((1, 1)),
]
_TC_OUT_SPEC = pl.BlockSpec((1, _BT), lambda i: (0, i))
_TC_SCRATCH = [pltpu.VMEM((384, _BT), jnp.float32)]


def _tc_call(interpret_args, *ops):
    return pl.pallas_call(
        _tc_body,
        grid=_TC_GRID,
        in_specs=_TC_IN_SPECS,
        out_specs=_TC_OUT_SPEC,
        out_shape=_TC_OUT_SHAPE,
        scratch_shapes=_TC_SCRATCH,
        **interpret_args,
    )(*ops)


def kernel(dense_x, lS_o, lS_i, emb,
           bot_W0, bot_b0, bot_W1, bot_b1, bot_W2, bot_b2,
           top_W0, top_b0, top_W1, top_b1, top_W2, top_b2):
    idx = lS_i.astype(jnp.int32)
    embT = jnp.swapaxes(emb, 1, 2)
    lyd = _sc_gather(idx, embT)

    w0t = bot_W0.T
    w1t = bot_W1.T
    w2t = bot_W2.T
    eye = jnp.eye(_BT, dtype=jnp.float32)
    wx = top_W0[:, :_M]
    wf = jnp.concatenate(
        [top_W0[:, _M:], jnp.zeros((512, 384 - 351), jnp.float32)],
        axis=1).astype(jnp.bfloat16)
    w4 = top_W1

    ops = (dense_x, lyd,
           w0t, bot_b0[None, :], w1t, bot_b1[None, :], w2t, bot_b2[None, :],
           eye, wx, wf, top_b0[:, None], w4, top_b1[:, None],
           top_W2.reshape(256, 1), top_b2[None, :])
    p = _tc_call({}, *ops)
    return p.reshape(_B, 1)
